# Initial kernel scaffold; baseline (speedup 1.0000x reference)
import jax
import jax.numpy as jnp
from jax.experimental import pallas as pl


def kernel(x, params):
    raise NotImplementedError("write your pallas kernel here")


# speedup vs baseline: 90.8869x; 90.8869x over previous
import jax
import jax.numpy as jnp
from jax.experimental import pallas as pl

E = 1024
H = 8
D = 8
C = 64
F32 = jnp.float32


def _leaky(a):
    return jnp.where(a >= 0.0, a, 0.2 * a)


def _dot(a, b):
    return jnp.dot(a, b, preferred_element_type=F32)


def _han_body(
    xA_ref, xR_ref,
    WA_ref, bA_ref, WE_ref, bE_ref, WR_ref, bR_ref, WW_ref, bW_ref,
    LsEA_ref, LdEA_ref, LsRE_ref, LdRE_ref, LsWE_ref, LdWE_ref,
    Wk_ref, bk_ref, q_ref,
    W2E_ref, b2E_ref, W2A_ref, b2A_ref, L2s_ref, L2d_ref,
    R8_ref,
    out_ref,
):
    xA = xA_ref[...]
    xR = xR_ref[...]
    R8 = R8_ref[...]

    xnA = _dot(xA, WA_ref[...]) + bA_ref[...]
    xnE = _dot(xR[:, 0:3], WE_ref[...]) + bE_ref[...]

    def slot_agg(col0, Wp_ref, bp_ref, Ls_ref, a_dst):
        Wp = Wp_ref[...]
        bp = bp_ref[...]
        Ls = Ls_ref[...]
        xns, alphas = [], []
        for j in range(10):
            feat = xR[:, col0 + 6 * j: col0 + 6 * j + 6]
            xnj = _dot(feat, Wp) + bp
            active = jnp.sum(feat, axis=1, keepdims=True) != 0.0
            a_src = _dot(xnj, Ls)
            al = _leaky(a_src + a_dst)
            al = jnp.where(active, al, -jnp.inf)
            xns.append(xnj)
            alphas.append(al)
        amax = alphas[0]
        for j in range(1, 10):
            amax = jnp.maximum(amax, alphas[j])
        amax = jnp.where(jnp.isfinite(amax), amax, 0.0)
        exs = [jnp.exp(a - amax) for a in alphas]
        s = exs[0]
        for j in range(1, 10):
            s = s + exs[j]
        inv = 1.0 / (s + 1e-16)
        agg = _dot(exs[0] * inv, R8) * xns[0]
        for j in range(1, 10):
            agg = agg + _dot(exs[j] * inv, R8) * xns[j]
        return jnp.maximum(agg, 0.0)

    ch_RE = slot_agg(3, WR_ref, bR_ref, LsRE_ref, _dot(xnE, LdRE_ref[...]))
    ch_WE = slot_agg(63, WW_ref, bW_ref, LsWE_ref, _dot(xnE, LdWE_ref[...]))
    ch_AE = jnp.maximum(xnA, 0.0)

    alEA = _leaky(_dot(xnE, LsEA_ref[...]) + _dot(xnA, LdEA_ref[...]))
    amax = jnp.max(alEA, axis=0, keepdims=True)
    ex = jnp.exp(alEA - amax)
    attn = ex / (jnp.sum(ex, axis=0, keepdims=True) + 1e-16)
    res1A = jnp.maximum(
        jnp.sum(_dot(attn, R8) * xnE, axis=0, keepdims=True), 0.0)

    Wk = Wk_ref[...]
    bk = bk_ref[...]
    q = q_ref[...]
    t0 = jnp.tanh(_dot(ch_AE, Wk) + bk)
    s0 = jnp.sum(t0 * q, axis=1, keepdims=True)
    t1 = jnp.mean(jnp.tanh(_dot(ch_RE, Wk) + bk), axis=0, keepdims=True)
    s1 = jnp.sum(t1 * q, axis=1, keepdims=True)
    t2 = jnp.mean(jnp.tanh(_dot(ch_WE, Wk) + bk), axis=0, keepdims=True)
    s2 = jnp.sum(t2 * q, axis=1, keepdims=True)
    m = jnp.maximum(jnp.maximum(s0, s1), s2)
    e0 = jnp.exp(s0 - m)
    e1 = jnp.exp(s1 - m)
    e2 = jnp.exp(s2 - m)
    invz = 1.0 / (e0 + e1 + e2)
    res1E = (e0 * invz) * ch_AE + (e1 * invz) * ch_RE + (e2 * invz) * ch_WE

    xn2E = _dot(res1E, W2E_ref[...]) + b2E_ref[...]
    xn2A = _dot(res1A, W2A_ref[...]) + b2A_ref[...]
    al2 = _leaky(_dot(xn2E, L2s_ref[...]) + _dot(xn2A, L2d_ref[...]))
    amax2 = jnp.max(al2, axis=0, keepdims=True)
    ex2 = jnp.exp(al2 - amax2)
    attn2 = ex2 / (jnp.sum(ex2, axis=0, keepdims=True) + 1e-16)
    agg2 = jnp.sum(_dot(attn2, R8) * xn2E, axis=0, keepdims=True)
    out_ref[...] = jnp.maximum(agg2, 0.0)


def _lmat(lin):
    eye = jnp.eye(H, dtype=lin.dtype)
    return (lin[:, :, None] * eye[:, None, :]).reshape(H * D, H)


def kernel(x, params):
    x_flat = x.reshape(-1)
    xA = x_flat[: 3 * E].reshape(1, 3 * E)
    xR = x_flat[3 * E:].reshape(E, 123)

    p1 = params['conv1']
    p2 = params['conv2']

    def row(v):
        return v.reshape(1, -1)

    R8 = jnp.kron(jnp.eye(H, dtype=F32), jnp.ones((1, D), F32))

    args = (
        xA, xR,
        p1['proj']['Arrived']['W'], row(p1['proj']['Arrived']['b']),
        p1['proj']['Expert']['W'], row(p1['proj']['Expert']['b']),
        p1['proj']['Running']['W'], row(p1['proj']['Running']['b']),
        p1['proj']['Waiting']['W'], row(p1['proj']['Waiting']['b']),
        _lmat(p1['lin_src']['Expert__Arrived']), _lmat(p1['lin_dst']['Expert__Arrived']),
        _lmat(p1['lin_src']['Running__Expert']), _lmat(p1['lin_dst']['Running__Expert']),
        _lmat(p1['lin_src']['Waiting__Expert']), _lmat(p1['lin_dst']['Waiting__Expert']),
        p1['k_lin']['W'], row(p1['k_lin']['b']), row(p1['q']),
        p2['proj']['Expert']['W'], row(p2['proj']['Expert']['b']),
        p2['proj']['Arrived']['W'], row(p2['proj']['Arrived']['b']),
        _lmat(p2['lin_src']['Expert__Arrived']), _lmat(p2['lin_dst']['Expert__Arrived']),
        R8,
    )

    return pl.pallas_call(
        _han_body,
        out_shape=jax.ShapeDtypeStruct((1, C), F32),
    )(*args)
